# baseline (speedup 1.0000x reference)
import functools

import jax
import jax.numpy as jnp
from jax import lax
from jax.experimental import pallas as pl
from jax.experimental.pallas import tpu as pltpu
from jax.experimental.pallas import tpu_sc as plsc

B = 16384
D = 64
NUM = 16
EPS = 1e-3

_NC, _NS = 2, 16
_NW = _NC * _NS
_BPW = B // _NW


def _sc_gather(idx_hbm, table_hbm, out_hbm, idx_v, rows_v, sem):
    wid = lax.axis_index("s") * _NC + lax.axis_index("c")
    base = wid * _BPW
    pltpu.sync_copy(idx_hbm.at[pl.ds(base, _BPW)], idx_v)
    pltpu.async_copy(table_hbm.at[idx_v], rows_v, sem).wait()
    pltpu.sync_copy(rows_v, out_hbm.at[pl.ds(base, _BPW)])


@functools.cache
def _gather_call():
    return functools.partial(
        pl.kernel,
        mesh=plsc.VectorSubcoreMesh(core_axis_name="c", subcore_axis_name="s"),
        out_type=jax.ShapeDtypeStruct((B, D), jnp.float32),
        compiler_params=pltpu.CompilerParams(use_tc_tiling_on_sc=False),
        scratch_types=[
            pltpu.VMEM((_BPW,), jnp.int32),
            pltpu.VMEM((_BPW, D), jnp.float32),
            pltpu.SemaphoreType.DMA,
        ],
    )(_sc_gather)


_BLK = 2048


def _mlp_body(emb, num, w1a, w1b, b1, w2, b2, w3, b3, out):
    h = jnp.maximum(emb[:] @ w1a[:] + num[:] @ w1b[:] + b1[:], 0.0)
    h = jnp.maximum(h @ w2[:] + b2[:], 0.0)
    o = h @ w3[:] + b3[:]
    sq = jnp.sum(o * o, axis=1, keepdims=True)
    out[:] = o * lax.rsqrt(jnp.maximum(sq, 1e-12))


_mlp_call = pl.pallas_call(
    _mlp_body,
    grid=(B // _BLK,),
    in_specs=[
        pl.BlockSpec((_BLK, D), lambda i: (i, 0)),
        pl.BlockSpec((_BLK, NUM), lambda i: (i, 0)),
        pl.BlockSpec((D, 128), lambda i: (0, 0)),
        pl.BlockSpec((NUM, 128), lambda i: (0, 0)),
        pl.BlockSpec((1, 128), lambda i: (0, 0)),
        pl.BlockSpec((128, 64), lambda i: (0, 0)),
        pl.BlockSpec((1, 64), lambda i: (0, 0)),
        pl.BlockSpec((64, D), lambda i: (0, 0)),
        pl.BlockSpec((1, D), lambda i: (0, 0)),
    ],
    out_specs=pl.BlockSpec((_BLK, D), lambda i: (i, 0)),
    out_shape=jax.ShapeDtypeStruct((B, D), jnp.float32),
)


def kernel(user_id, user_numerical_features, table, W1, b1, gamma1, beta1,
           W2, b2, gamma2, beta2, W3, b3):
    idx = user_id.astype(jnp.int32)
    emb = _gather_call()(idx, table)

    s = jax.lax.rsqrt(jnp.float32(1.0 + EPS))
    g1 = s * gamma1
    W2f = g1[:, None] * W2
    b2f = b2 + beta1 @ W2
    g2 = s * gamma2
    W3f = g2[:, None] * W3
    b3f = b3 + beta2 @ W3

    out = _mlp_call(
        emb,
        user_numerical_features,
        W1[:D, :],
        W1[D:, :],
        b1[None, :],
        W2f,
        b2f[None, :],
        W3f,
        b3f[None, :],
    )
    return out


# speedup vs baseline: 1.6145x; 1.6145x over previous
import functools

import jax
import jax.numpy as jnp
from jax import lax
from jax.experimental import pallas as pl
from jax.experimental.pallas import tpu as pltpu
from jax.experimental.pallas import tpu_sc as plsc

B = 16384
D = 64
NUM = 16
EPS = 1e-3

_NC, _NS = 2, 16
_NW = _NC * _NS
_BPW = B // _NW
_FIRE = 16


def _sc_gather(idx_hbm, table_hbm, out_hbm, idx_v, idx_s, rows_v, sem):
    wid = lax.axis_index("s") * _NC + lax.axis_index("c")
    base = wid * _BPW
    pltpu.sync_copy(idx_hbm.at[pl.ds(base, _BPW)], idx_v)

    def batch(b, carry):
        v16 = idx_v[pl.ds(b * _FIRE, _FIRE)]
        descs = []
        for k in range(_FIRE):
            i = b * _FIRE + k
            descs.append(pltpu.async_copy(
                table_hbm.at[pl.ds(v16[k], 1)], rows_v.at[pl.ds(i, 1)], sem))
        for d in descs:
            d.wait()
        return carry

    lax.fori_loop(0, _BPW // _FIRE, batch, 0)
    pltpu.sync_copy(rows_v, out_hbm.at[pl.ds(base, _BPW)])


@functools.cache
def _gather_call():
    return functools.partial(
        pl.kernel,
        mesh=plsc.VectorSubcoreMesh(core_axis_name="c", subcore_axis_name="s"),
        out_type=jax.ShapeDtypeStruct((B, D), jnp.float32),
        compiler_params=pltpu.CompilerParams(use_tc_tiling_on_sc=True),
        scratch_types=[
            pltpu.VMEM((_BPW,), jnp.int32),
            pltpu.SMEM((_BPW,), jnp.int32),
            pltpu.VMEM((_BPW, D), jnp.float32),
            pltpu.SemaphoreType.DMA,
        ],
    )(_sc_gather)


_BLK = 2048


def _mlp_body(emb, num, w1, b1, g1, be1, w2, b2, g2, be2, w3, b3, out):
    s = lax.rsqrt(jnp.float32(1.0 + EPS))
    w1full = w1[...]
    h = jnp.maximum(emb[...] @ w1full[:D] + num[...] @ w1full[D:] + b1[...], 0.0)
    h = h * (s * g1[...]) + be1[...]
    h = jnp.maximum(h @ w2[...] + b2[...], 0.0)
    h = h * (s * g2[...]) + be2[...]
    o = h @ w3[...] + b3[...]
    sq = jnp.sum(o * o, axis=1, keepdims=True)
    out[...] = o * lax.rsqrt(jnp.maximum(sq, 1e-12))


_mlp_call = pl.pallas_call(
    _mlp_body,
    grid=(B // _BLK,),
    in_specs=[
        pl.BlockSpec((_BLK, D), lambda i: (i, 0)),
        pl.BlockSpec((_BLK, NUM), lambda i: (i, 0)),
        pl.BlockSpec((D + NUM, 128), lambda i: (0, 0)),
        pl.BlockSpec((128,), lambda i: (0,)),
        pl.BlockSpec((128,), lambda i: (0,)),
        pl.BlockSpec((128,), lambda i: (0,)),
        pl.BlockSpec((128, 64), lambda i: (0, 0)),
        pl.BlockSpec((64,), lambda i: (0,)),
        pl.BlockSpec((64,), lambda i: (0,)),
        pl.BlockSpec((64,), lambda i: (0,)),
        pl.BlockSpec((64, D), lambda i: (0, 0)),
        pl.BlockSpec((D,), lambda i: (0,)),
    ],
    out_specs=pl.BlockSpec((_BLK, D), lambda i: (i, 0)),
    out_shape=jax.ShapeDtypeStruct((B, D), jnp.float32),
)


def kernel(user_id, user_numerical_features, table, W1, b1, gamma1, beta1,
           W2, b2, gamma2, beta2, W3, b3):
    idx = user_id.astype(jnp.int32)
    emb = _gather_call()(idx, table)
    return _mlp_call(emb, user_numerical_features, W1, b1, gamma1, beta1,
                     W2, b2, gamma2, beta2, W3, b3)
